# baseline (speedup 1.0000x reference)
import functools

import jax
import jax.numpy as jnp
from jax import lax
from jax.experimental import pallas as pl
from jax.experimental.pallas import tpu as pltpu
from jax.experimental.pallas import tpu_sc as plsc

_EPS = 1e-6


def _make_sc_kernel(B, V, D, n_cores, n_subcores):
    nw = n_cores * n_subcores
    b_per_w = B // nw
    n_grp = b_per_w // 16
    mesh = plsc.VectorSubcoreMesh(core_axis_name="c", subcore_axis_name="s")

    @functools.partial(
        pl.kernel,
        mesh=mesh,
        compiler_params=pltpu.CompilerParams(needs_layout_passes=False),
        out_type=[
            jax.ShapeDtypeStruct((B * D,), jnp.float32),
            jax.ShapeDtypeStruct((B * D,), jnp.float32),
        ],
        scratch_types=[
            pltpu.VMEM((b_per_w,), jnp.int32),
            pltpu.VMEM((b_per_w,), jnp.int32),
            pltpu.VMEM((128, 128), jnp.float32),
            pltpu.VMEM((b_per_w * D,), jnp.float32),
            pltpu.VMEM((b_per_w * D,), jnp.float32),
            pltpu.SemaphoreType.DMA,
        ],
    )
    def sc_kernel(idx_hbm, table_hbm, pz_hbm, z_hbm,
                  idx_v, gidx_v, grp_v, pz_v, z_v, sem):
        wid = lax.axis_index("s") * n_cores + lax.axis_index("c")
        base = wid * b_per_w
        pltpu.sync_copy(idx_hbm.at[pl.ds(base, b_per_w)], idx_v)

        def mk_gidx(j, carry):
            v = idx_v[pl.ds(j * 16, 16)]
            gidx_v[pl.ds(j * 16, 16)] = lax.shift_right_logical(v, 3)
            return carry

        lax.fori_loop(0, n_grp, mk_gidx, 0)

        lanes = lax.iota(jnp.int32, 16)
        n_chunks = b_per_w // 128

        def chunk_body(c, carry):
            cbase = c * 128
            pltpu.async_copy(table_hbm.at[gidx_v.at[pl.ds(cbase, 128)]],
                             grp_v, sem).wait()

            def body(j, carry2):
                rowbase = cbase + j * 16
                idxvec = idx_v[pl.ds(rowbase, 16)]
                offs = lax.shift_left(idxvec & 7, 4)
                rows = j * 16 + lanes
                flatbase = lax.shift_left(rowbase + lanes, 4)
                for f in range(D):
                    colf = offs + f
                    x = plsc.load_gather(grp_v, [rows, colf])
                    p = 1.0 / (1.0 + jnp.exp(-x))
                    p = p * (1.0 - 2.0 * _EPS) + _EPS
                    flat = flatbase + f
                    plsc.store_scatter(pz_v, [flat], p)
                    zval = jnp.where(p > 0.5, 1.0, 0.0)
                    plsc.store_scatter(z_v, [flat], zval)
                return carry2

            lax.fori_loop(0, 8, body, 0)
            return carry

        lax.fori_loop(0, n_chunks, chunk_body, 0)

        pltpu.sync_copy(pz_v, pz_hbm.at[pl.ds(base * D, b_per_w * D)])
        pltpu.sync_copy(z_v, z_hbm.at[pl.ds(base * D, b_per_w * D)])

    return sc_kernel


def kernel(inputs, embedding):
    B = inputs.shape[0]
    V, D = embedding.shape
    info = plsc.get_sparse_core_info()
    idx = inputs.reshape(-1).astype(jnp.int32)
    table = embedding.reshape(V // 8, 8 * D)
    sc = _make_sc_kernel(B, V, D, info.num_cores, info.num_subcores)
    pz, z_f = sc(idx, table)
    return pz.reshape(B, D), z_f.astype(jnp.bool_).reshape(B, D)
